# Initial kernel scaffold; baseline (speedup 1.0000x reference)
import jax
import jax.numpy as jnp
from jax.experimental import pallas as pl


def kernel(output, target):
    raise NotImplementedError("write your pallas kernel here")


# speedup vs baseline: 1.6835x; 1.6835x over previous
import functools
import math

import jax
import jax.numpy as jnp
from jax.experimental import pallas as pl

SMOOTHING = 0.1
N_CLASSES = 100000
CONFIDENCE = 1.0 - SMOOTHING
SMOOTHING_VALUE = SMOOTHING / (N_CLASSES - 1)

ROW_BLOCK = 8


def _loss_kernel(t_ref, x_ref, o_ref):
    i = pl.program_id(0)
    x = x_ref[...]
    t = t_ref[0]
    m = jnp.max(x, axis=1, keepdims=True)
    s = jnp.sum(jnp.exp(x - m), axis=1, keepdims=True)
    lse = m + jnp.log(s)
    row_sum = jnp.sum(x, axis=1, keepdims=True)
    cols = jax.lax.broadcasted_iota(jnp.int32, x.shape, 1)
    g = jnp.sum(jnp.where(cols == t, x, 0.0), axis=1, keepdims=True)
    partial = jnp.sum(
        lse
        - SMOOTHING_VALUE * row_sum
        - (CONFIDENCE - SMOOTHING_VALUE) * g
    ).reshape(1, 1)

    @pl.when(i == 0)
    def _init():
        B = ROW_BLOCK * pl.num_programs(0)
        const = B * (
            (N_CLASSES - 1) * SMOOTHING_VALUE * math.log(SMOOTHING_VALUE)
            + CONFIDENCE * math.log(CONFIDENCE)
        )
        o_ref[...] = jnp.full((1, 1), const, dtype=jnp.float32)

    o_ref[...] += partial


@functools.partial(jax.jit, static_argnames=("interpret",))
def kernel(output, target, interpret=False):
    B, N = output.shape
    n_blocks = B // ROW_BLOCK
    t3 = target.astype(jnp.int32).reshape(n_blocks, ROW_BLOCK, 1)
    out = pl.pallas_call(
        _loss_kernel,
        grid=(n_blocks,),
        in_specs=[
            pl.BlockSpec((1, ROW_BLOCK, 1), lambda i: (i, 0, 0)),
            pl.BlockSpec((ROW_BLOCK, N), lambda i: (i, 0)),
        ],
        out_specs=pl.BlockSpec((1, 1), lambda i: (0, 0)),
        out_shape=jax.ShapeDtypeStruct((1, 1), jnp.float32),
        interpret=interpret,
    )(t3, output)
    return out[0, 0]


# speedup vs baseline: 2.0247x; 1.2027x over previous
import functools
import math

import jax
import jax.numpy as jnp
from jax.experimental import pallas as pl

SMOOTHING = 0.1
N_CLASSES = 100000
CONFIDENCE = 1.0 - SMOOTHING
SMOOTHING_VALUE = SMOOTHING / (N_CLASSES - 1)

ROW_BLOCK = 32


def _loss_kernel(t_ref, x_ref, o_ref):
    i = pl.program_id(0)
    x = x_ref[...]
    t = t_ref[0]
    m = jnp.max(x, axis=1, keepdims=True)
    s = jnp.sum(jnp.exp(x - m), axis=1, keepdims=True)
    lse = m + jnp.log(s)
    row_sum = jnp.sum(x, axis=1, keepdims=True)
    cols = jax.lax.broadcasted_iota(jnp.int32, x.shape, 1)
    g = jnp.sum(jnp.where(cols == t, x, 0.0), axis=1, keepdims=True)
    partial = jnp.sum(
        lse
        - SMOOTHING_VALUE * row_sum
        - (CONFIDENCE - SMOOTHING_VALUE) * g
    ).reshape(1, 1)

    @pl.when(i == 0)
    def _init():
        B = ROW_BLOCK * pl.num_programs(0)
        const = B * (
            (N_CLASSES - 1) * SMOOTHING_VALUE * math.log(SMOOTHING_VALUE)
            + CONFIDENCE * math.log(CONFIDENCE)
        )
        o_ref[...] = jnp.full((1, 1), const, dtype=jnp.float32)

    o_ref[...] += partial


@functools.partial(jax.jit, static_argnames=("interpret",))
def kernel(output, target, interpret=False):
    B, N = output.shape
    n_blocks = B // ROW_BLOCK
    t3 = target.astype(jnp.int32).reshape(n_blocks, ROW_BLOCK, 1)
    out = pl.pallas_call(
        _loss_kernel,
        grid=(n_blocks,),
        in_specs=[
            pl.BlockSpec((1, ROW_BLOCK, 1), lambda i: (i, 0, 0)),
            pl.BlockSpec((ROW_BLOCK, N), lambda i: (i, 0)),
        ],
        out_specs=pl.BlockSpec((1, 1), lambda i: (0, 0)),
        out_shape=jax.ShapeDtypeStruct((1, 1), jnp.float32),
        interpret=interpret,
    )(t3, output)
    return out[0, 0]
